# baseline (speedup 1.0000x reference)
import functools
import math

import jax
import jax.numpy as jnp
from jax import lax
from jax.experimental import pallas as pl
from jax.experimental.pallas import tpu as pltpu
from jax.experimental.pallas import tpu_sc as plsc

D_EMB = 64
LANES = 16
UNIT = 128
G = 4
SCALE = math.sqrt(D_EMB)


@functools.partial(jax.jit, static_argnames=("n_units",))
def _emb_lookup(xf, table, n_units):
    info = plsc.get_sparse_core_info()
    nc, ns = info.num_cores, info.num_subcores
    nw = nc * ns
    units_per_w = n_units // nw
    groups_per_w = units_per_w // G
    n_rows = n_units * UNIT

    mesh = plsc.VectorSubcoreMesh(core_axis_name="c", subcore_axis_name="s")

    @functools.partial(
        pl.kernel,
        mesh=mesh,
        out_type=jax.ShapeDtypeStruct((n_rows, D_EMB), jnp.float32),
        scratch_types=[
            pltpu.VMEM((G, UNIT), jnp.int32),
            pltpu.VMEM((G * UNIT, D_EMB), jnp.float32),
            pltpu.SemaphoreType.DMA,
        ],
        compiler_params=pltpu.CompilerParams(use_tc_tiling_on_sc=False),
    )
    def k(x_hbm, table_hbm, out_hbm, idx_v, rows_v, sem):
        wid = lax.axis_index("s") * nc + lax.axis_index("c")
        unit_base = wid * units_per_w

        def group_body(g, carry):
            unit0 = unit_base + g * G
            pltpu.sync_copy(x_hbm.at[pl.ds(unit0, G)], idx_v)
            handles = []
            for u in range(G):
                handles.append(
                    pltpu.async_copy(
                        table_hbm.at[idx_v.at[u]],
                        rows_v.at[pl.ds(u * UNIT, UNIT)],
                        sem,
                    )
                )
            for h in handles:
                h.wait()

            def scale_body(r, c2):
                base = r * 4
                for rr in range(4):
                    for c in range(D_EMB // LANES):
                        sl = pl.ds(c * LANES, LANES)
                        rows_v[base + rr, sl] = rows_v[base + rr, sl] * SCALE
                return c2

            lax.fori_loop(0, G * UNIT // 4, scale_body, 0)
            pltpu.sync_copy(rows_v, out_hbm.at[pl.ds(unit0 * UNIT, G * UNIT)])
            return carry

        lax.fori_loop(0, groups_per_w, group_body, 0)

    return k(xf, table)


def kernel(x, table):
    b0, b1 = x.shape
    n_rows = b0 * b1
    n_units = n_rows // UNIT
    xf = x.reshape(n_units, UNIT)
    out = _emb_lookup(xf, table, n_units)
    return out.reshape(b0, b1, D_EMB)
